# Initial kernel scaffold; baseline (speedup 1.0000x reference)
import jax
import jax.numpy as jnp
from jax.experimental import pallas as pl


def kernel(X, steps, A):
    raise NotImplementedError("write your pallas kernel here")


# speedup vs baseline: 1.3459x; 1.3459x over previous
import jax
import jax.numpy as jnp
from jax.experimental import pallas as pl
from jax.experimental.pallas import tpu as pltpu

_BM = 512


def _step_body(a_ref, h_ref, hself_ref, out_ref):
    a = a_ref[...]
    deg = jnp.sum(a, axis=1, keepdims=True)
    neigh = jnp.dot(
        a.astype(jnp.bfloat16),
        h_ref[...].astype(jnp.bfloat16),
        preferred_element_type=jnp.float32,
    )
    h = hself_ref[...] + neigh / (deg + 1e-10)
    norm = jnp.sqrt(jnp.sum(h * h, axis=1, keepdims=True))
    out_ref[...] = h / (norm + 1e-10)


def _gs_step(h, A):
    N, D = h.shape
    nm = N // _BM
    return pl.pallas_call(
        _step_body,
        grid=(nm,),
        in_specs=[
            pl.BlockSpec((_BM, N), lambda m: (m, 0)),
            pl.BlockSpec((N, D), lambda m: (0, 0)),
            pl.BlockSpec((_BM, D), lambda m: (m, 0)),
        ],
        out_specs=pl.BlockSpec((_BM, D), lambda m: (m, 0)),
        out_shape=jax.ShapeDtypeStruct((N, D), jnp.float32),
    )(A, h, h)


def kernel(X, steps, A):
    return jax.lax.fori_loop(0, steps, lambda _, h: _gs_step(h, A), X)
